# Initial kernel scaffold; baseline (speedup 1.0000x reference)
import jax
import jax.numpy as jnp
from jax.experimental import pallas as pl


def kernel(x):
    raise NotImplementedError("write your pallas kernel here")


# speedup vs baseline: 2.6432x; 2.6432x over previous
import functools

import jax
import jax.numpy as jnp
from jax import lax
from jax.experimental import pallas as pl
from jax.experimental.pallas import tpu as pltpu

MASK_RATIO = 0.5


def _energy_body(x_ref, e_ref, *, inv_c):
    e_ref[...] = (jnp.sum(x_ref[0], axis=0) * inv_c)[None, None, None]


def _select_body(e_ref, m_ref, key_ref, *, k, hw, idx_bits):
    b = e_ref.shape[0]
    e = e_ref[...]
    bu = lax.bitcast_convert_type(e, jnp.uint32)
    key = jnp.where(bu >= jnp.uint32(0x80000000), ~bu, bu | jnp.uint32(0x80000000))
    key_ref[...] = key

    def p1(_, carry):
        t, bm = carry
        cand = t | bm
        cnt = jnp.sum((key_ref[...] >= cand).astype(jnp.int32), axis=1,
                      keepdims=True)
        return jnp.where(cnt >= k, cand, t), bm >> 1

    thr, _ = lax.fori_loop(
        0, 32, p1,
        (jnp.zeros((b, 1), jnp.uint32), jnp.full((b, 1), 0x80000000, jnp.uint32)))

    kk = key_ref[...]
    gt = kk > thr
    eq = kk == thr
    cnt_gt = jnp.sum(gt.astype(jnp.int32), axis=1, keepdims=True)
    need = k - cnt_gt

    iota = lax.broadcasted_iota(jnp.int32, (b, hw), 1)

    def p2(_, carry):
        t, bm = carry
        cand = t | bm
        cnt = jnp.sum((eq & (iota < cand)).astype(jnp.int32), axis=1,
                      keepdims=True)
        return jnp.where(cnt < need, cand, t), bm >> 1

    mt, _ = lax.fori_loop(
        0, idx_bits, p2,
        (jnp.zeros((b, 1), jnp.int32),
         jnp.full((b, 1), 1 << (idx_bits - 1), jnp.int32)))

    mask = gt | (eq & (iota < mt + 1))
    m_ref[...] = mask.astype(jnp.float32)


def _apply_body(x_ref, m_ref, o_ref):
    o_ref[...] = x_ref[...] * m_ref[0, 0]


def kernel(x):
    b, c, h, w = x.shape
    hw = h * w
    k = int(hw * MASK_RATIO)
    schunks = 8
    sblk = hw // schunks
    xf = x.reshape(b, c, hw)

    energy = pl.pallas_call(
        functools.partial(_energy_body, inv_c=1.0 / c),
        grid=(b, schunks),
        in_specs=[pl.BlockSpec((1, c, sblk), lambda i, s: (i, 0, s))],
        out_specs=pl.BlockSpec((1, 1, 1, sblk), lambda i, s: (i, s, 0, 0)),
        out_shape=jax.ShapeDtypeStruct((b, schunks, 1, sblk), jnp.float32),
    )(xf).reshape(b, hw)

    idx_bits = max(1, (hw - 1).bit_length())
    mask = pl.pallas_call(
        functools.partial(_select_body, k=k, hw=hw, idx_bits=idx_bits),
        in_specs=[pl.BlockSpec((b, hw), lambda: (0, 0))],
        out_specs=pl.BlockSpec((b, hw), lambda: (0, 0)),
        out_shape=jax.ShapeDtypeStruct((b, hw), jnp.float32),
        scratch_shapes=[pltpu.VMEM((b, hw), jnp.uint32)],
    )(energy)

    mask4 = mask.reshape(b, schunks, 1, sblk)
    out = pl.pallas_call(
        _apply_body,
        grid=(b, schunks),
        in_specs=[
            pl.BlockSpec((1, c, sblk), lambda i, s: (i, 0, s)),
            pl.BlockSpec((1, 1, 1, sblk), lambda i, s: (i, s, 0, 0)),
        ],
        out_specs=pl.BlockSpec((1, c, sblk), lambda i, s: (i, 0, s)),
        out_shape=jax.ShapeDtypeStruct((b, c, hw), jnp.float32),
    )(xf, mask4)

    return out.reshape(b, c, h, w)


# speedup vs baseline: 8.1309x; 3.0762x over previous
import functools

import jax
import jax.numpy as jnp
from jax import lax
from jax.experimental import pallas as pl
from jax.experimental.pallas import tpu as pltpu

MASK_RATIO = 0.5


def _energy_body(x_ref, e_ref, *, inv_c):
    e_ref[...] = (jnp.sum(x_ref[0], axis=0) * inv_c)[None]


def _select_body(e_ref, m_ref, key_ref, *, k, idx_bits):
    b, h, w = e_ref.shape
    e = e_ref[...]
    bu = lax.bitcast_convert_type(e, jnp.uint32)
    key = jnp.where(bu >= jnp.uint32(0x80000000), ~bu, bu | jnp.uint32(0x80000000))
    key_ref[...] = key

    def p1(_, carry):
        t, bm = carry
        cand = t | bm
        cnt = jnp.sum((key_ref[...] >= cand).astype(jnp.int32), axis=(1, 2),
                      keepdims=True)
        return jnp.where(cnt >= k, cand, t), bm >> 1

    thr, _ = lax.fori_loop(
        0, 32, p1,
        (jnp.zeros((b, 1, 1), jnp.uint32),
         jnp.full((b, 1, 1), 0x80000000, jnp.uint32)))

    kk = key_ref[...]
    gt = kk > thr
    eq = kk == thr
    cnt_gt = jnp.sum(gt.astype(jnp.int32), axis=(1, 2), keepdims=True)
    need = k - cnt_gt

    idx = (lax.broadcasted_iota(jnp.int32, (b, h, w), 1) * w
           + lax.broadcasted_iota(jnp.int32, (b, h, w), 2))

    def p2(_, carry):
        t, bm = carry
        cand = t | bm
        cnt = jnp.sum((eq & (idx < cand)).astype(jnp.int32), axis=(1, 2),
                      keepdims=True)
        return jnp.where(cnt < need, cand, t), bm >> 1

    mt, _ = lax.fori_loop(
        0, idx_bits, p2,
        (jnp.zeros((b, 1, 1), jnp.int32),
         jnp.full((b, 1, 1), 1 << (idx_bits - 1), jnp.int32)))

    mask = gt | (eq & (idx < mt + 1))
    m_ref[...] = mask.astype(jnp.float32)


def _apply_body(x_ref, m_ref, o_ref):
    o_ref[...] = x_ref[...] * m_ref[...][:, None]


def kernel(x):
    b, c, h, w = x.shape
    hw = h * w
    k = int(hw * MASK_RATIO)
    hchunks = 8
    hblk = h // hchunks

    energy = pl.pallas_call(
        functools.partial(_energy_body, inv_c=1.0 / c),
        grid=(b, hchunks),
        in_specs=[pl.BlockSpec((1, c, hblk, w), lambda i, s: (i, 0, s, 0))],
        out_specs=pl.BlockSpec((1, hblk, w), lambda i, s: (i, s, 0)),
        out_shape=jax.ShapeDtypeStruct((b, h, w), jnp.float32),
    )(x)

    idx_bits = max(1, (hw - 1).bit_length())
    mask = pl.pallas_call(
        functools.partial(_select_body, k=k, idx_bits=idx_bits),
        in_specs=[pl.BlockSpec((b, h, w), lambda: (0, 0, 0))],
        out_specs=pl.BlockSpec((b, h, w), lambda: (0, 0, 0)),
        out_shape=jax.ShapeDtypeStruct((b, h, w), jnp.float32),
        scratch_shapes=[pltpu.VMEM((b, h, w), jnp.uint32)],
    )(energy)

    out = pl.pallas_call(
        _apply_body,
        grid=(b, hchunks),
        in_specs=[
            pl.BlockSpec((1, c, hblk, w), lambda i, s: (i, 0, s, 0)),
            pl.BlockSpec((1, hblk, w), lambda i, s: (i, s, 0)),
        ],
        out_specs=pl.BlockSpec((1, c, hblk, w), lambda i, s: (i, 0, s, 0)),
        out_shape=jax.ShapeDtypeStruct((b, c, h, w), jnp.float32),
    )(x, mask)

    return out
